# baseline (speedup 1.0000x reference)
import jax
import jax.numpy as jnp
from jax.experimental import pallas as pl
from jax.experimental.pallas import tpu as pltpu

DEPTH = 1000
N = 51200
DEPTH_PER_BLOCK = 40


def _one_hot_t_block(x_hbm, out_ref, x_vmem, sem):
    i = pl.program_id(0)

    @pl.when(i == 0)
    def _load():
        pltpu.async_copy(x_hbm, x_vmem, sem).wait()

    d0 = i * DEPTH_PER_BLOCK
    x = x_vmem[:].astype(jnp.int32)
    drow = jax.lax.broadcasted_iota(jnp.int32, (DEPTH_PER_BLOCK, N), 0) + d0
    out_ref[:] = (drow == x).astype(jnp.float32)


def kernel(x):
    x_row = jnp.reshape(x, (1, N))
    num_blocks = DEPTH // DEPTH_PER_BLOCK
    out_t = pl.pallas_call(
        _one_hot_t_block,
        grid=(num_blocks,),
        in_specs=[pl.BlockSpec(memory_space=pl.ANY)],
        out_specs=pl.BlockSpec((DEPTH_PER_BLOCK, N), lambda i: (i, 0)),
        out_shape=jax.ShapeDtypeStruct((DEPTH, N), jnp.float32),
        scratch_shapes=[
            pltpu.VMEM((1, N), jnp.float32),
            pltpu.SemaphoreType.DMA,
        ],
    )(x_row)
    return jnp.reshape(jnp.transpose(out_t), (1, N, DEPTH))


# speedup vs baseline: 1.0001x; 1.0001x over previous
import jax
import jax.numpy as jnp
from jax.experimental import pallas as pl

DEPTH = 1000
DEPTH_PER_BLOCK = 40


def _one_hot_t_block(idx_ref, out_ref):
    d0 = pl.program_id(0) * DEPTH_PER_BLOCK
    idx = idx_ref[:].astype(jnp.int32)
    n = idx_ref.shape[1]
    drow = jax.lax.broadcasted_iota(jnp.int32, (DEPTH_PER_BLOCK, n), 0) + d0
    out_ref[:] = (drow == idx).astype(jnp.float32)


def kernel(x):
    n = x.size
    x_row = jnp.reshape(x, (1, n))
    num_blocks = DEPTH // DEPTH_PER_BLOCK
    out_t = pl.pallas_call(
        _one_hot_t_block,
        grid=(num_blocks,),
        in_specs=[pl.BlockSpec((1, n), lambda i: (0, 0))],
        out_specs=pl.BlockSpec((DEPTH_PER_BLOCK, n), lambda i: (i, 0)),
        out_shape=jax.ShapeDtypeStruct((DEPTH, n), jnp.float32),
    )(x_row)
    return jnp.reshape(jnp.transpose(out_t), (1, n, DEPTH))
